# Initial kernel scaffold; baseline (speedup 1.0000x reference)
import jax
import jax.numpy as jnp
from jax.experimental import pallas as pl


def kernel(img):
    raise NotImplementedError("write your pallas kernel here")


# speedup vs baseline: 1.0156x; 1.0156x over previous
import jax
import jax.numpy as jnp
from jax.experimental import pallas as pl

_C, _H, _W = 96, 512, 512
_BR = 32


def _select_block(x_ref, o_ref):
    x = x_ref[...]
    mask = x[_C - 1 : _C, :, :] > 0.9
    o_ref[...] = jnp.where(mask, x, 0.0)


def kernel(img):
    return pl.pallas_call(
        _select_block,
        grid=(_H // _BR,),
        in_specs=[pl.BlockSpec((_C, _BR, _W), lambda i: (0, i, 0))],
        out_specs=pl.BlockSpec((_C, _BR, _W), lambda i: (0, i, 0)),
        out_shape=jax.ShapeDtypeStruct((_C, _H, _W), jnp.float32),
    )(img)


# speedup vs baseline: 1.0162x; 1.0006x over previous
import jax
import jax.numpy as jnp
from jax.experimental import pallas as pl

_C, _H, _W = 96, 512, 512
_BC = 8


def _select_block(x_ref, m_ref, o_ref):
    mask = m_ref[...] > 0.9
    o_ref[...] = jnp.where(mask, x_ref[...], 0.0)


def kernel(img):
    return pl.pallas_call(
        _select_block,
        grid=(_C // _BC,),
        in_specs=[
            pl.BlockSpec((_BC, _H, _W), lambda i: (i, 0, 0)),
            pl.BlockSpec((1, _H, _W), lambda i: (_C - 1, 0, 0)),
        ],
        out_specs=pl.BlockSpec((_BC, _H, _W), lambda i: (i, 0, 0)),
        out_shape=jax.ShapeDtypeStruct((_C, _H, _W), jnp.float32),
    )(img, img)
